# baseline (speedup 1.0000x reference)
import functools

import jax
import jax.numpy as jnp
from jax import lax
from jax.experimental import pallas as pl
from jax.experimental.pallas import tpu as pltpu
from jax.experimental.pallas import tpu_sc as plsc

DIM = 128
HEADS = 4
HD = DIM // HEADS
NPAIR = 2
PW = DIM // NPAIR
ACCW = 128
N_NODES = 10000
N_EDGES = 320000
N_TILES = 16
E_PER_TILE = N_EDGES // N_TILES
EB = 80
NBLK = E_PER_TILE // EB
INV_SQRT_HD = 1.0 / (HD ** 0.5)


def _mm_body(x_ref, w_ref, b_ref, o_ref, *, act):
    y = jnp.dot(x_ref[...], w_ref[...], preferred_element_type=jnp.float32)
    y = y + b_ref[...]
    if act:
        y = y * jax.nn.sigmoid(y)
    o_ref[...] = y


def _matmul(x, w, b, act=False, bn=1000):
    n, k = x.shape
    m = w.shape[1]
    return pl.pallas_call(
        functools.partial(_mm_body, act=act),
        grid=(n // bn,),
        in_specs=[
            pl.BlockSpec((bn, k), lambda i: (i, 0)),
            pl.BlockSpec((k, m), lambda i: (0, 0)),
            pl.BlockSpec((1, m), lambda i: (0, 0)),
        ],
        out_specs=pl.BlockSpec((bn, m), lambda i: (i, 0)),
        out_shape=jax.ShapeDtypeStruct((n, m), jnp.float32),
    )(x, w, b.reshape(1, m))


def _att_from_acc(acc_ref):
    parts = []
    for c in range(NPAIR):
        d0 = acc_ref[c, :, PW:PW + 1] + 1e-16
        d1 = acc_ref[c, :, PW + 1:PW + 2] + 1e-16
        parts.append(acc_ref[c, :, 0:HD] / d0)
        parts.append(acc_ref[c, :, HD:PW] / d1)
    return jnp.concatenate(parts, axis=-1)


def _combine_body(*refs, nrel):
    o_ref = refs[-1]
    g_ref, bt_ref = refs[-3], refs[-2]
    x = jnp.zeros_like(o_ref)
    for r in range(nrel):
        x = x + _att_from_acc(refs[2 * r]) + refs[2 * r + 1][...]
    mu = jnp.mean(x, axis=-1, keepdims=True)
    var = jnp.mean((x - mu) ** 2, axis=-1, keepdims=True)
    y = (x - mu) / jnp.sqrt(var + 1e-5) * g_ref[...] + bt_ref[...]
    o_ref[...] = y * jax.nn.sigmoid(y)


def _combine(accs, skips, gamma, beta, bn=1000):
    nrel = len(accs)
    n = skips[0].shape[0]
    ops, specs = [], []
    for acc, skip in zip(accs, skips):
        ops += [acc, skip]
        specs += [pl.BlockSpec((NPAIR, bn, ACCW), lambda i: (0, i, 0)),
                  pl.BlockSpec((bn, DIM), lambda i: (i, 0))]
    ops += [gamma.reshape(1, DIM), beta.reshape(1, DIM)]
    specs += [pl.BlockSpec((1, DIM), lambda i: (0, 0)),
              pl.BlockSpec((1, DIM), lambda i: (0, 0))]
    return pl.pallas_call(
        functools.partial(_combine_body, nrel=nrel),
        grid=(n // bn,),
        in_specs=specs,
        out_specs=pl.BlockSpec((bn, DIM), lambda i: (i, 0)),
        out_shape=jax.ShapeDtypeStruct((n, DIM), jnp.float32),
    )(*ops)


def _bcast_sum(a, lane):
    dnums = lax.GatherDimensionNumbers(
        offset_dims=(), collapsed_slice_dims=(0,), start_index_map=(0,))
    for k in (1, 2, 4, 8):
        idx = jnp.bitwise_xor(lane, k)
        a = a + lax.gather(a, idx.reshape(16, 1), dnums, (1,),
                           mode=lax.GatherScatterMode.PROMISE_IN_BOUNDS)
    return a


def _edge_compute(qv, kvv, mv, qoff, lane):
    def edge(e, _):
        q0 = qv[e, pl.ds(qoff, 16)]
        q1 = qv[e, pl.ds(qoff + 16, 16)]
        q2 = qv[e, pl.ds(qoff + 32, 16)]
        q3 = qv[e, pl.ds(qoff + 48, 16)]
        k0 = kvv[e, pl.ds(0, 16)]
        k1 = kvv[e, pl.ds(16, 16)]
        k2 = kvv[e, pl.ds(32, 16)]
        k3 = kvv[e, pl.ds(48, 16)]
        e0 = jnp.exp(_bcast_sum(q0 * k0 + q1 * k1, lane))
        e1 = jnp.exp(_bcast_sum(q2 * k2 + q3 * k3, lane))
        v0 = kvv[e, pl.ds(64, 16)]
        v1 = kvv[e, pl.ds(80, 16)]
        v2 = kvv[e, pl.ds(96, 16)]
        v3 = kvv[e, pl.ds(112, 16)]
        mv[e, pl.ds(0, 16)] = v0 * e0
        mv[e, pl.ds(16, 16)] = v1 * e0
        mv[e, pl.ds(32, 16)] = v2 * e1
        mv[e, pl.ds(48, 16)] = v3 * e1
        exrow = jnp.where(lane == 0, e0, jnp.where(lane == 1, e1,
                          jnp.zeros((16,), jnp.float32)))
        mv[e, pl.ds(64, 16)] = exrow
        return 0
    lax.fori_loop(0, EB, edge, 0)


def _edge_body(q_hbm, kv_hbm, src_hbm, dst_hbm, acc_hbm,
               srcv, dstv, qv, kvv, mv, zb, acc_sh, sem):
    c = lax.axis_index("c")
    s = lax.axis_index("s")
    lane = lax.iota(jnp.int32, 16)

    def zrow(i, _):
        for j in range(ACCW // 16):
            zb[i, pl.ds(16 * j, 16)] = jnp.zeros((16,), jnp.float32)
            mv[i, pl.ds(16 * j, 16)] = jnp.zeros((16,), jnp.float32)
        return 0
    lax.fori_loop(0, EB, zrow, 0)
    base_row = s * 624
    for i in range(7):
        pltpu.sync_copy(zb, acc_sh.at[pl.ds(base_row + i * EB, EB)])

    @pl.when(s == N_TILES - 1)
    def _():
        pltpu.sync_copy(zb, acc_sh.at[pl.ds(base_row + 7 * EB, EB)])

    @pl.when(s != N_TILES - 1)
    def _():
        pltpu.sync_copy(zb.at[pl.ds(0, 64)],
                        acc_sh.at[pl.ds(base_row + 7 * EB, 64)])
    plsc.subcore_barrier()

    qoff = c * PW
    chunk = s * E_PER_TILE

    def blk(b, _):
        off = chunk + b * EB
        pltpu.sync_copy(src_hbm.at[pl.ds(off, EB)], srcv)
        pltpu.sync_copy(dst_hbm.at[pl.ds(off, EB)], dstv)
        pltpu.async_copy(q_hbm.at[dstv], qv, sem).wait()
        pltpu.async_copy(kv_hbm.at[c].at[srcv], kvv, sem).wait()
        _edge_compute(qv, kvv, mv, qoff, lane)
        pltpu.sync_copy(mv, acc_sh.at[dstv], add=True)
        return 0

    lax.fori_loop(0, NBLK, blk, 0)
    plsc.subcore_barrier()

    @pl.when(s != N_TILES - 1)
    def _():
        pltpu.sync_copy(acc_sh.at[pl.ds(base_row, 624)],
                        acc_hbm.at[c].at[pl.ds(base_row, 624)])

    @pl.when(s == N_TILES - 1)
    def _():
        pltpu.sync_copy(acc_sh.at[pl.ds(base_row, 640)],
                        acc_hbm.at[c].at[pl.ds(base_row, 640)])


def _edge_conv(q_tab, kv_tab, src, dst):
    mesh = plsc.VectorSubcoreMesh(core_axis_name="c", subcore_axis_name="s")
    f = pl.kernel(
        _edge_body,
        out_type=jax.ShapeDtypeStruct((NPAIR, N_NODES, ACCW), jnp.float32),
        mesh=mesh,
        scratch_types=[
            pltpu.VMEM((EB,), jnp.int32),
            pltpu.VMEM((EB,), jnp.int32),
            pltpu.VMEM((EB, DIM), jnp.float32),
            pltpu.VMEM((EB, 2 * PW), jnp.float32),
            pltpu.VMEM((EB, ACCW), jnp.float32),
            pltpu.VMEM((EB, ACCW), jnp.float32),
            pltpu.VMEM_SHARED((N_NODES, ACCW), jnp.float32),
            pltpu.SemaphoreType.DMA,
        ],
    )
    return f(q_tab, kv_tab, src, dst)


def _split_tables(y, off):
    return y[:, off:off + DIM] * INV_SQRT_HD


def _kv_tables(k, v):
    return jnp.stack([jnp.concatenate([k[:, :PW], v[:, :PW]], axis=1),
                      jnp.concatenate([k[:, PW:], v[:, PW:]], axis=1)])


def kernel(x_curr, x_act, ei_c2a, ei_a2c, ei_a2a, params):
    p = params
    edges = {
        'c2a': (ei_c2a[0].astype(jnp.int32), ei_c2a[1].astype(jnp.int32)),
        'a2c': (ei_a2c[0].astype(jnp.int32), ei_a2c[1].astype(jnp.int32)),
        'a2a': (ei_a2a[0].astype(jnp.int32), ei_a2a[1].astype(jnp.int32)),
    }

    xc = _matmul(x_curr, p['in_curr'][0], p['in_curr'][1], act=True)
    xa = _matmul(x_act, p['in_act'][0], p['in_act'][1], act=True)

    for lp in p['layers']:
        wc = jnp.concatenate([lp['c2a']['k'][0], lp['c2a']['v'][0],
                              lp['a2c']['q'][0], lp['a2c']['skip'][0]], axis=1)
        bc = jnp.concatenate([lp['c2a']['k'][1], lp['c2a']['v'][1],
                              lp['a2c']['q'][1], lp['a2c']['skip'][1]])
        yc = _matmul(xc, wc, bc)
        wa = jnp.concatenate([lp['c2a']['q'][0], lp['c2a']['skip'][0],
                              lp['a2c']['k'][0], lp['a2c']['v'][0],
                              lp['a2a']['q'][0], lp['a2a']['k'][0],
                              lp['a2a']['v'][0], lp['a2a']['skip'][0]], axis=1)
        ba = jnp.concatenate([lp['c2a']['q'][1], lp['c2a']['skip'][1],
                              lp['a2c']['k'][1], lp['a2c']['v'][1],
                              lp['a2a']['q'][1], lp['a2a']['k'][1],
                              lp['a2a']['v'][1], lp['a2a']['skip'][1]])
        ya = _matmul(xa, wa, ba)

        q_c2a = _split_tables(ya, 0)
        skip_c2a = ya[:, DIM:2 * DIM]
        kv_c2a = _kv_tables(yc[:, 0:DIM], yc[:, DIM:2 * DIM])
        q_a2c = _split_tables(yc, 2 * DIM)
        skip_a2c = yc[:, 3 * DIM:4 * DIM]
        kv_a2c = _kv_tables(ya[:, 2 * DIM:3 * DIM], ya[:, 3 * DIM:4 * DIM])
        q_a2a = _split_tables(ya, 4 * DIM)
        skip_a2a = ya[:, 7 * DIM:8 * DIM]
        kv_a2a = _kv_tables(ya[:, 5 * DIM:6 * DIM], ya[:, 6 * DIM:7 * DIM])

        acc_c2a = _edge_conv(q_c2a, kv_c2a, *edges['c2a'])
        acc_a2a = _edge_conv(q_a2a, kv_a2a, *edges['a2a'])
        acc_a2c = _edge_conv(q_a2c, kv_a2c, *edges['a2c'])

        xa = _combine([acc_c2a, acc_a2a], [skip_c2a, skip_a2a],
                      p['norm_act'][0], p['norm_act'][1])
        xc = _combine([acc_a2c], [skip_a2c],
                      p['norm_curr'][0], p['norm_curr'][1])

    return _matmul(xa, p['out_act'][0], p['out_act'][1])


# speedup vs baseline: 1.1497x; 1.1497x over previous
import functools

import jax
import jax.numpy as jnp
from jax import lax
from jax.experimental import pallas as pl
from jax.experimental.pallas import tpu as pltpu
from jax.experimental.pallas import tpu_sc as plsc

DIM = 128
HEADS = 4
HD = DIM // HEADS
NPAIR = 2
PW = DIM // NPAIR
ACCW = 128
N_NODES = 10000
N_EDGES = 320000
N_TILES = 16
E_PER_TILE = N_EDGES // N_TILES
EB = 80
NBLK = E_PER_TILE // EB
INV_SQRT_HD = 1.0 / (HD ** 0.5)


def _mm_body(x_ref, w_ref, b_ref, o_ref, *, act):
    y = jnp.dot(x_ref[...], w_ref[...], preferred_element_type=jnp.float32)
    y = y + b_ref[...]
    if act:
        y = y * jax.nn.sigmoid(y)
    o_ref[...] = y


def _matmul(x, w, b, act=False, bn=1000):
    n, k = x.shape
    m = w.shape[1]
    return pl.pallas_call(
        functools.partial(_mm_body, act=act),
        grid=(n // bn,),
        in_specs=[
            pl.BlockSpec((bn, k), lambda i: (i, 0)),
            pl.BlockSpec((k, m), lambda i: (0, 0)),
            pl.BlockSpec((1, m), lambda i: (0, 0)),
        ],
        out_specs=pl.BlockSpec((bn, m), lambda i: (i, 0)),
        out_shape=jax.ShapeDtypeStruct((n, m), jnp.float32),
    )(x, w, b.reshape(1, m))


def _att_from_acc(acc_ref):
    parts = []
    for c in range(NPAIR):
        d0 = acc_ref[c, :, PW:PW + 1] + 1e-16
        d1 = acc_ref[c, :, PW + 1:PW + 2] + 1e-16
        parts.append(acc_ref[c, :, 0:HD] / d0)
        parts.append(acc_ref[c, :, HD:PW] / d1)
    return jnp.concatenate(parts, axis=-1)


def _combine_body(*refs, nrel):
    o_ref = refs[-1]
    g_ref, bt_ref = refs[-3], refs[-2]
    x = jnp.zeros_like(o_ref)
    for r in range(nrel):
        x = x + _att_from_acc(refs[2 * r]) + refs[2 * r + 1][...]
    mu = jnp.mean(x, axis=-1, keepdims=True)
    var = jnp.mean((x - mu) ** 2, axis=-1, keepdims=True)
    y = (x - mu) / jnp.sqrt(var + 1e-5) * g_ref[...] + bt_ref[...]
    o_ref[...] = y * jax.nn.sigmoid(y)


def _combine(accs, skips, gamma, beta, bn=1000):
    nrel = len(accs)
    n = skips[0].shape[0]
    ops, specs = [], []
    for acc, skip in zip(accs, skips):
        ops += [acc, skip]
        specs += [pl.BlockSpec((NPAIR, bn, ACCW), lambda i: (0, i, 0)),
                  pl.BlockSpec((bn, DIM), lambda i: (i, 0))]
    ops += [gamma.reshape(1, DIM), beta.reshape(1, DIM)]
    specs += [pl.BlockSpec((1, DIM), lambda i: (0, 0)),
              pl.BlockSpec((1, DIM), lambda i: (0, 0))]
    return pl.pallas_call(
        functools.partial(_combine_body, nrel=nrel),
        grid=(n // bn,),
        in_specs=specs,
        out_specs=pl.BlockSpec((bn, DIM), lambda i: (i, 0)),
        out_shape=jax.ShapeDtypeStruct((n, DIM), jnp.float32),
    )(*ops)


def _bcast_sum(a, lane):
    dnums = lax.GatherDimensionNumbers(
        offset_dims=(), collapsed_slice_dims=(0,), start_index_map=(0,))
    for k in (1, 2, 4, 8):
        idx = jnp.bitwise_xor(lane, k)
        a = a + lax.gather(a, idx.reshape(16, 1), dnums, (1,),
                           mode=lax.GatherScatterMode.PROMISE_IN_BOUNDS)
    return a


def _edge_compute(qv, kvv, mv, qoff, lane):
    def edge(e, _):
        q0 = qv[e, pl.ds(qoff, 16)]
        q1 = qv[e, pl.ds(qoff + 16, 16)]
        q2 = qv[e, pl.ds(qoff + 32, 16)]
        q3 = qv[e, pl.ds(qoff + 48, 16)]
        k0 = kvv[e, pl.ds(0, 16)]
        k1 = kvv[e, pl.ds(16, 16)]
        k2 = kvv[e, pl.ds(32, 16)]
        k3 = kvv[e, pl.ds(48, 16)]
        e0 = jnp.exp(_bcast_sum(q0 * k0 + q1 * k1, lane))
        e1 = jnp.exp(_bcast_sum(q2 * k2 + q3 * k3, lane))
        v0 = kvv[e, pl.ds(64, 16)]
        v1 = kvv[e, pl.ds(80, 16)]
        v2 = kvv[e, pl.ds(96, 16)]
        v3 = kvv[e, pl.ds(112, 16)]
        mv[e, pl.ds(0, 16)] = v0 * e0
        mv[e, pl.ds(16, 16)] = v1 * e0
        mv[e, pl.ds(32, 16)] = v2 * e1
        mv[e, pl.ds(48, 16)] = v3 * e1
        exrow = jnp.where(lane == 0, e0, jnp.where(lane == 1, e1,
                          jnp.zeros((16,), jnp.float32)))
        mv[e, pl.ds(64, 16)] = exrow
        return 0
    lax.fori_loop(0, EB, edge, 0)


def _edge_body(q_hbm, kv_hbm, src_hbm, dst_hbm, acc_hbm,
               srcv, dstv, qv, kvv, mv, zb, acc_sh, sem, semk):
    c = lax.axis_index("c")
    s = lax.axis_index("s")
    lane = lax.iota(jnp.int32, 16)

    def zrow(i, _):
        for j in range(ACCW // 16):
            zb[i, pl.ds(16 * j, 16)] = jnp.zeros((16,), jnp.float32)
            mv[i, pl.ds(16 * j, 16)] = jnp.zeros((16,), jnp.float32)
        return 0
    lax.fori_loop(0, EB, zrow, 0)
    base_row = s * 624
    for i in range(7):
        pltpu.sync_copy(zb, acc_sh.at[pl.ds(base_row + i * EB, EB)])

    @pl.when(s == N_TILES - 1)
    def _():
        pltpu.sync_copy(zb, acc_sh.at[pl.ds(base_row + 7 * EB, EB)])

    @pl.when(s != N_TILES - 1)
    def _():
        pltpu.sync_copy(zb.at[pl.ds(0, 64)],
                        acc_sh.at[pl.ds(base_row + 7 * EB, 64)])
    plsc.subcore_barrier()

    qoff = c * PW
    chunk = s * E_PER_TILE

    def blk(b, _):
        off = chunk + b * EB
        pltpu.sync_copy(src_hbm.at[pl.ds(off, EB)], srcv)
        pltpu.sync_copy(dst_hbm.at[pl.ds(off, EB)], dstv)
        cq = pltpu.async_copy(q_hbm.at[dstv], qv, sem)
        ck = pltpu.async_copy(kv_hbm.at[c].at[srcv], kvv, semk)
        cq.wait()
        ck.wait()
        _edge_compute(qv, kvv, mv, qoff, lane)
        pltpu.sync_copy(mv, acc_sh.at[dstv], add=True)
        return 0

    lax.fori_loop(0, NBLK, blk, 0)
    plsc.subcore_barrier()

    @pl.when(s != N_TILES - 1)
    def _():
        pltpu.sync_copy(acc_sh.at[pl.ds(base_row, 624)],
                        acc_hbm.at[c].at[pl.ds(base_row, 624)])

    @pl.when(s == N_TILES - 1)
    def _():
        pltpu.sync_copy(acc_sh.at[pl.ds(base_row, 640)],
                        acc_hbm.at[c].at[pl.ds(base_row, 640)])


def _edge_conv(q_tab, kv_tab, src, dst):
    mesh = plsc.VectorSubcoreMesh(core_axis_name="c", subcore_axis_name="s")
    f = pl.kernel(
        _edge_body,
        out_type=jax.ShapeDtypeStruct((NPAIR, N_NODES, ACCW), jnp.float32),
        mesh=mesh,
        scratch_types=[
            pltpu.VMEM((EB,), jnp.int32),
            pltpu.VMEM((EB,), jnp.int32),
            pltpu.VMEM((EB, DIM), jnp.float32),
            pltpu.VMEM((EB, 2 * PW), jnp.float32),
            pltpu.VMEM((EB, ACCW), jnp.float32),
            pltpu.VMEM((EB, ACCW), jnp.float32),
            pltpu.VMEM_SHARED((N_NODES, ACCW), jnp.float32),
            pltpu.SemaphoreType.DMA,
            pltpu.SemaphoreType.DMA,
        ],
    )
    return f(q_tab, kv_tab, src, dst)


def _split_tables(y, off):
    return y[:, off:off + DIM] * INV_SQRT_HD


def _kv_tables(k, v):
    return jnp.stack([jnp.concatenate([k[:, :PW], v[:, :PW]], axis=1),
                      jnp.concatenate([k[:, PW:], v[:, PW:]], axis=1)])


def kernel(x_curr, x_act, ei_c2a, ei_a2c, ei_a2a, params):
    p = params
    edges = {
        'c2a': (ei_c2a[0].astype(jnp.int32), ei_c2a[1].astype(jnp.int32)),
        'a2c': (ei_a2c[0].astype(jnp.int32), ei_a2c[1].astype(jnp.int32)),
        'a2a': (ei_a2a[0].astype(jnp.int32), ei_a2a[1].astype(jnp.int32)),
    }

    xc = _matmul(x_curr, p['in_curr'][0], p['in_curr'][1], act=True)
    xa = _matmul(x_act, p['in_act'][0], p['in_act'][1], act=True)

    for lp in p['layers']:
        wc = jnp.concatenate([lp['c2a']['k'][0], lp['c2a']['v'][0],
                              lp['a2c']['q'][0], lp['a2c']['skip'][0]], axis=1)
        bc = jnp.concatenate([lp['c2a']['k'][1], lp['c2a']['v'][1],
                              lp['a2c']['q'][1], lp['a2c']['skip'][1]])
        yc = _matmul(xc, wc, bc)
        wa = jnp.concatenate([lp['c2a']['q'][0], lp['c2a']['skip'][0],
                              lp['a2c']['k'][0], lp['a2c']['v'][0],
                              lp['a2a']['q'][0], lp['a2a']['k'][0],
                              lp['a2a']['v'][0], lp['a2a']['skip'][0]], axis=1)
        ba = jnp.concatenate([lp['c2a']['q'][1], lp['c2a']['skip'][1],
                              lp['a2c']['k'][1], lp['a2c']['v'][1],
                              lp['a2a']['q'][1], lp['a2a']['k'][1],
                              lp['a2a']['v'][1], lp['a2a']['skip'][1]])
        ya = _matmul(xa, wa, ba)

        q_c2a = _split_tables(ya, 0)
        skip_c2a = ya[:, DIM:2 * DIM]
        kv_c2a = _kv_tables(yc[:, 0:DIM], yc[:, DIM:2 * DIM])
        q_a2c = _split_tables(yc, 2 * DIM)
        skip_a2c = yc[:, 3 * DIM:4 * DIM]
        kv_a2c = _kv_tables(ya[:, 2 * DIM:3 * DIM], ya[:, 3 * DIM:4 * DIM])
        q_a2a = _split_tables(ya, 4 * DIM)
        skip_a2a = ya[:, 7 * DIM:8 * DIM]
        kv_a2a = _kv_tables(ya[:, 5 * DIM:6 * DIM], ya[:, 6 * DIM:7 * DIM])

        acc_c2a = _edge_conv(q_c2a, kv_c2a, *edges['c2a'])
        acc_a2a = _edge_conv(q_a2a, kv_a2a, *edges['a2a'])
        acc_a2c = _edge_conv(q_a2c, kv_a2c, *edges['a2c'])

        xa = _combine([acc_c2a, acc_a2a], [skip_c2a, skip_a2a],
                      p['norm_act'][0], p['norm_act'][1])
        xc = _combine([acc_a2c], [skip_a2c],
                      p['norm_curr'][0], p['norm_curr'][1])

    return _matmul(xa, p['out_act'][0], p['out_act'][1])


# speedup vs baseline: 1.1998x; 1.0435x over previous
import functools

import jax
import jax.numpy as jnp
from jax import lax
from jax.experimental import pallas as pl
from jax.experimental.pallas import tpu as pltpu
from jax.experimental.pallas import tpu_sc as plsc

DIM = 128
HEADS = 4
HD = DIM // HEADS
NPAIR = 2
PW = DIM // NPAIR
ACCW = 128
N_NODES = 10000
N_EDGES = 320000
N_TILES = 16
E_PER_TILE = N_EDGES // N_TILES
EB = 80
NBLK = E_PER_TILE // EB
INV_SQRT_HD = 1.0 / (HD ** 0.5)


def _mm_body(x_ref, w_ref, b_ref, o_ref, *, act):
    y = jnp.dot(x_ref[...], w_ref[...], preferred_element_type=jnp.float32)
    y = y + b_ref[...]
    if act:
        y = y * jax.nn.sigmoid(y)
    o_ref[...] = y


def _matmul(x, w, b, act=False, bn=1000):
    n, k = x.shape
    m = w.shape[1]
    return pl.pallas_call(
        functools.partial(_mm_body, act=act),
        grid=(n // bn,),
        in_specs=[
            pl.BlockSpec((bn, k), lambda i: (i, 0)),
            pl.BlockSpec((k, m), lambda i: (0, 0)),
            pl.BlockSpec((1, m), lambda i: (0, 0)),
        ],
        out_specs=pl.BlockSpec((bn, m), lambda i: (i, 0)),
        out_shape=jax.ShapeDtypeStruct((n, m), jnp.float32),
    )(x, w, b.reshape(1, m))


def _att_from_acc(acc_ref):
    parts = []
    for c in range(NPAIR):
        d0 = acc_ref[c, :, PW:PW + 1] + 1e-16
        d1 = acc_ref[c, :, PW + 1:PW + 2] + 1e-16
        parts.append(acc_ref[c, :, 0:HD] / d0)
        parts.append(acc_ref[c, :, HD:PW] / d1)
    return jnp.concatenate(parts, axis=-1)


def _combine_body(*refs, nrel):
    o_ref = refs[-1]
    g_ref, bt_ref = refs[-3], refs[-2]
    x = jnp.zeros_like(o_ref)
    for r in range(nrel):
        x = x + _att_from_acc(refs[2 * r]) + refs[2 * r + 1][...]
    mu = jnp.mean(x, axis=-1, keepdims=True)
    var = jnp.mean((x - mu) ** 2, axis=-1, keepdims=True)
    y = (x - mu) / jnp.sqrt(var + 1e-5) * g_ref[...] + bt_ref[...]
    o_ref[...] = y * jax.nn.sigmoid(y)


def _combine(accs, skips, gamma, beta, bn=1000):
    nrel = len(accs)
    n = skips[0].shape[0]
    ops, specs = [], []
    for acc, skip in zip(accs, skips):
        ops += [acc, skip]
        specs += [pl.BlockSpec((NPAIR, bn, ACCW), lambda i: (0, i, 0)),
                  pl.BlockSpec((bn, DIM), lambda i: (i, 0))]
    ops += [gamma.reshape(1, DIM), beta.reshape(1, DIM)]
    specs += [pl.BlockSpec((1, DIM), lambda i: (0, 0)),
              pl.BlockSpec((1, DIM), lambda i: (0, 0))]
    return pl.pallas_call(
        functools.partial(_combine_body, nrel=nrel),
        grid=(n // bn,),
        in_specs=specs,
        out_specs=pl.BlockSpec((bn, DIM), lambda i: (i, 0)),
        out_shape=jax.ShapeDtypeStruct((n, DIM), jnp.float32),
    )(*ops)


def _bcast_sum(a, lane):
    dnums = lax.GatherDimensionNumbers(
        offset_dims=(), collapsed_slice_dims=(0,), start_index_map=(0,))
    for k in (1, 2, 4, 8):
        idx = jnp.bitwise_xor(lane, k)
        a = a + lax.gather(a, idx.reshape(16, 1), dnums, (1,),
                           mode=lax.GatherScatterMode.PROMISE_IN_BOUNDS)
    return a


def _edge_compute(qv, kvv, mv, qoff, lane):
    def edge(e, _):
        q0 = qv[e, pl.ds(qoff, 16)]
        q1 = qv[e, pl.ds(qoff + 16, 16)]
        q2 = qv[e, pl.ds(qoff + 32, 16)]
        q3 = qv[e, pl.ds(qoff + 48, 16)]
        k0 = kvv[e, pl.ds(0, 16)]
        k1 = kvv[e, pl.ds(16, 16)]
        k2 = kvv[e, pl.ds(32, 16)]
        k3 = kvv[e, pl.ds(48, 16)]
        e0 = jnp.exp(_bcast_sum(q0 * k0 + q1 * k1, lane))
        e1 = jnp.exp(_bcast_sum(q2 * k2 + q3 * k3, lane))
        v0 = kvv[e, pl.ds(64, 16)]
        v1 = kvv[e, pl.ds(80, 16)]
        v2 = kvv[e, pl.ds(96, 16)]
        v3 = kvv[e, pl.ds(112, 16)]
        mv[e, pl.ds(0, 16)] = v0 * e0
        mv[e, pl.ds(16, 16)] = v1 * e0
        mv[e, pl.ds(32, 16)] = v2 * e1
        mv[e, pl.ds(48, 16)] = v3 * e1
        exrow = jnp.where(lane == 0, e0, jnp.where(lane == 1, e1,
                          jnp.zeros((16,), jnp.float32)))
        mv[e, pl.ds(64, 16)] = exrow
        return 0
    lax.fori_loop(0, EB, edge, 0)


def _edge_body(q_hbm, kv_hbm, src_hbm, dst_hbm, acc_hbm,
               srcv, dstv, srcv2, dstv2, qv, kvv, mv, zb, acc_sh,
               sem, semk, sems):
    c = lax.axis_index("c")
    s = lax.axis_index("s")
    lane = lax.iota(jnp.int32, 16)

    def zrow(i, _):
        for j in range(ACCW // 16):
            zb[i, pl.ds(16 * j, 16)] = jnp.zeros((16,), jnp.float32)
            mv[i, pl.ds(16 * j, 16)] = jnp.zeros((16,), jnp.float32)
        return 0
    lax.fori_loop(0, EB, zrow, 0)
    base_row = s * 624
    for i in range(7):
        pltpu.sync_copy(zb, acc_sh.at[pl.ds(base_row + i * EB, EB)])

    @pl.when(s == N_TILES - 1)
    def _():
        pltpu.sync_copy(zb, acc_sh.at[pl.ds(base_row + 7 * EB, EB)])

    @pl.when(s != N_TILES - 1)
    def _():
        pltpu.sync_copy(zb.at[pl.ds(0, 64)],
                        acc_sh.at[pl.ds(base_row + 7 * EB, 64)])
    plsc.subcore_barrier()

    qoff = c * PW
    chunk = s * E_PER_TILE

    def blk2(i, _):
        off0 = chunk + (2 * i) * EB
        off1 = off0 + EB
        pltpu.sync_copy(src_hbm.at[pl.ds(off0, EB)], srcv)
        pltpu.sync_copy(dst_hbm.at[pl.ds(off0, EB)], dstv)
        cq = pltpu.async_copy(q_hbm.at[dstv], qv, sem)
        ck = pltpu.async_copy(kv_hbm.at[c].at[srcv], kvv, semk)
        cq.wait()
        ck.wait()
        _edge_compute(qv, kvv, mv, qoff, lane)
        sc0 = pltpu.async_copy(mv, acc_sh.at[dstv], sems, add=True)
        pltpu.sync_copy(src_hbm.at[pl.ds(off1, EB)], srcv2)
        pltpu.sync_copy(dst_hbm.at[pl.ds(off1, EB)], dstv2)
        cq = pltpu.async_copy(q_hbm.at[dstv2], qv, sem)
        ck = pltpu.async_copy(kv_hbm.at[c].at[srcv2], kvv, semk)
        cq.wait()
        ck.wait()
        _edge_compute(qv, kvv, zb, qoff, lane)
        sc1 = pltpu.async_copy(zb, acc_sh.at[dstv2], sems, add=True)
        sc0.wait()
        sc1.wait()
        return 0

    lax.fori_loop(0, NBLK // 2, blk2, 0)
    plsc.subcore_barrier()

    @pl.when(s != N_TILES - 1)
    def _():
        pltpu.sync_copy(acc_sh.at[pl.ds(base_row, 624)],
                        acc_hbm.at[c].at[pl.ds(base_row, 624)])

    @pl.when(s == N_TILES - 1)
    def _():
        pltpu.sync_copy(acc_sh.at[pl.ds(base_row, 640)],
                        acc_hbm.at[c].at[pl.ds(base_row, 640)])


def _edge_conv(q_tab, kv_tab, src, dst):
    mesh = plsc.VectorSubcoreMesh(core_axis_name="c", subcore_axis_name="s")
    f = pl.kernel(
        _edge_body,
        out_type=jax.ShapeDtypeStruct((NPAIR, N_NODES, ACCW), jnp.float32),
        mesh=mesh,
        scratch_types=[
            pltpu.VMEM((EB,), jnp.int32),
            pltpu.VMEM((EB,), jnp.int32),
            pltpu.VMEM((EB,), jnp.int32),
            pltpu.VMEM((EB,), jnp.int32),
            pltpu.VMEM((EB, DIM), jnp.float32),
            pltpu.VMEM((EB, 2 * PW), jnp.float32),
            pltpu.VMEM((EB, ACCW), jnp.float32),
            pltpu.VMEM((EB, ACCW), jnp.float32),
            pltpu.VMEM_SHARED((N_NODES, ACCW), jnp.float32),
            pltpu.SemaphoreType.DMA,
            pltpu.SemaphoreType.DMA,
            pltpu.SemaphoreType.DMA,
        ],
    )
    return f(q_tab, kv_tab, src, dst)


def _split_tables(y, off):
    return y[:, off:off + DIM] * INV_SQRT_HD


def _kv_tables(k, v):
    return jnp.stack([jnp.concatenate([k[:, :PW], v[:, :PW]], axis=1),
                      jnp.concatenate([k[:, PW:], v[:, PW:]], axis=1)])


def kernel(x_curr, x_act, ei_c2a, ei_a2c, ei_a2a, params):
    p = params
    edges = {
        'c2a': (ei_c2a[0].astype(jnp.int32), ei_c2a[1].astype(jnp.int32)),
        'a2c': (ei_a2c[0].astype(jnp.int32), ei_a2c[1].astype(jnp.int32)),
        'a2a': (ei_a2a[0].astype(jnp.int32), ei_a2a[1].astype(jnp.int32)),
    }

    xc = _matmul(x_curr, p['in_curr'][0], p['in_curr'][1], act=True)
    xa = _matmul(x_act, p['in_act'][0], p['in_act'][1], act=True)

    for lp in p['layers']:
        wc = jnp.concatenate([lp['c2a']['k'][0], lp['c2a']['v'][0],
                              lp['a2c']['q'][0], lp['a2c']['skip'][0]], axis=1)
        bc = jnp.concatenate([lp['c2a']['k'][1], lp['c2a']['v'][1],
                              lp['a2c']['q'][1], lp['a2c']['skip'][1]])
        yc = _matmul(xc, wc, bc)
        wa = jnp.concatenate([lp['c2a']['q'][0], lp['c2a']['skip'][0],
                              lp['a2c']['k'][0], lp['a2c']['v'][0],
                              lp['a2a']['q'][0], lp['a2a']['k'][0],
                              lp['a2a']['v'][0], lp['a2a']['skip'][0]], axis=1)
        ba = jnp.concatenate([lp['c2a']['q'][1], lp['c2a']['skip'][1],
                              lp['a2c']['k'][1], lp['a2c']['v'][1],
                              lp['a2a']['q'][1], lp['a2a']['k'][1],
                              lp['a2a']['v'][1], lp['a2a']['skip'][1]])
        ya = _matmul(xa, wa, ba)

        q_c2a = _split_tables(ya, 0)
        skip_c2a = ya[:, DIM:2 * DIM]
        kv_c2a = _kv_tables(yc[:, 0:DIM], yc[:, DIM:2 * DIM])
        q_a2c = _split_tables(yc, 2 * DIM)
        skip_a2c = yc[:, 3 * DIM:4 * DIM]
        kv_a2c = _kv_tables(ya[:, 2 * DIM:3 * DIM], ya[:, 3 * DIM:4 * DIM])
        q_a2a = _split_tables(ya, 4 * DIM)
        skip_a2a = ya[:, 7 * DIM:8 * DIM]
        kv_a2a = _kv_tables(ya[:, 5 * DIM:6 * DIM], ya[:, 6 * DIM:7 * DIM])

        acc_c2a = _edge_conv(q_c2a, kv_c2a, *edges['c2a'])
        acc_a2a = _edge_conv(q_a2a, kv_a2a, *edges['a2a'])
        acc_a2c = _edge_conv(q_a2c, kv_a2c, *edges['a2c'])

        xa = _combine([acc_c2a, acc_a2a], [skip_c2a, skip_a2a],
                      p['norm_act'][0], p['norm_act'][1])
        xc = _combine([acc_a2c], [skip_a2c],
                      p['norm_curr'][0], p['norm_curr'][1])

    return _matmul(xa, p['out_act'][0], p['out_act'][1])
